# Initial kernel scaffold; baseline (speedup 1.0000x reference)
import jax
import jax.numpy as jnp
from jax.experimental import pallas as pl


def kernel(x, router_input, W1, b1, W2, b2, W3, b3, Wr, br):
    raise NotImplementedError("write your pallas kernel here")


# speedup vs baseline: 4.4229x; 4.4229x over previous
import functools

import jax
import jax.numpy as jnp
from jax.experimental import pallas as pl

E = 8
TOP_K = 2
C_IN = 96
HID = 96
C_OUT = 96
R_IN = 8

TILE = 1024


def _gelu(x):
    return 0.5 * x * (1.0 + jax.lax.erf(x * 0.7071067811865476))


def _moe_body(x_ref, r_ref, w1_ref, b1_ref, w2_ref, b2_ref, w3_ref, b3_ref,
              wr_ref, br_ref, o_ref):
    xt = x_ref[0]
    rt = r_ref[0]
    t = xt.shape[-1]

    logits = jnp.dot(wr_ref[...], rt, preferred_element_type=jnp.float32)
    logits = logits + br_ref[...].reshape(E, 1)

    eidx = jax.lax.broadcasted_iota(jnp.int32, (E, t), 0)
    v1 = jnp.max(logits, axis=0, keepdims=True)
    idx1 = jnp.min(jnp.where(logits == v1, eidx, E), axis=0, keepdims=True)
    sel1 = eidx == idx1
    rest = jnp.where(sel1, -jnp.inf, logits)
    v2 = jnp.max(rest, axis=0, keepdims=True)
    idx2 = jnp.min(jnp.where(rest == v2, eidx, E), axis=0, keepdims=True)
    sel2 = eidx == idx2
    g1 = 1.0 / (1.0 + jnp.exp(v2 - v1))
    g2 = 1.0 - g1
    gates = jnp.where(sel1, g1, 0.0) + jnp.where(sel2, g2, 0.0)

    acc = jnp.zeros((C_OUT, t), dtype=jnp.float32)
    for e in range(E):
        h = jnp.dot(w1_ref[e], xt, preferred_element_type=jnp.float32)
        h = _gelu(h + b1_ref[e][:, None])
        h = jnp.dot(w2_ref[e], h, preferred_element_type=jnp.float32)
        h = _gelu(h + b2_ref[e][:, None])
        h = jnp.dot(w3_ref[e], h, preferred_element_type=jnp.float32)
        h = h + b3_ref[e][:, None]
        acc = acc + gates[e][None, :] * h
    o_ref[0] = acc


@functools.partial(jax.jit, static_argnames=())
def kernel(x, router_input, W1, b1, W2, b2, W3, b3, Wr, br):
    B, _, H, W = x.shape
    n = H * W
    xr = x.reshape(B, C_IN, n)
    rr = router_input.reshape(B, R_IN, n)
    nt = n // TILE

    grid = (B, nt)
    out = pl.pallas_call(
        _moe_body,
        grid=grid,
        in_specs=[
            pl.BlockSpec((1, C_IN, TILE), lambda b, i: (b, 0, i)),
            pl.BlockSpec((1, R_IN, TILE), lambda b, i: (b, 0, i)),
            pl.BlockSpec((E, HID, C_IN), lambda b, i: (0, 0, 0)),
            pl.BlockSpec((E, HID), lambda b, i: (0, 0)),
            pl.BlockSpec((E, HID, HID), lambda b, i: (0, 0, 0)),
            pl.BlockSpec((E, HID), lambda b, i: (0, 0)),
            pl.BlockSpec((E, C_OUT, HID), lambda b, i: (0, 0, 0)),
            pl.BlockSpec((E, C_OUT), lambda b, i: (0, 0)),
            pl.BlockSpec((E, R_IN), lambda b, i: (0, 0)),
            pl.BlockSpec((1, E), lambda b, i: (0, 0)),
        ],
        out_specs=pl.BlockSpec((1, C_OUT, TILE), lambda b, i: (b, 0, i)),
        out_shape=jax.ShapeDtypeStruct((B, C_OUT, n), jnp.float32),
    )(xr, rr, W1, b1, W2, b2, W3, b3, Wr, br.reshape(1, E))
    return out.reshape(B, C_OUT, H, W)
